# Initial kernel scaffold; baseline (speedup 1.0000x reference)
import jax
import jax.numpy as jnp
from jax.experimental import pallas as pl


def kernel(adj, features, W):
    raise NotImplementedError("write your pallas kernel here")


# speedup vs baseline: 2.0009x; 2.0009x over previous
import jax
import jax.numpy as jnp
from jax import lax
from jax.experimental import pallas as pl
from jax.experimental.pallas import tpu as pltpu

_N = 10000
_D = 128
_BM = 400


def _sage_body(adj_ref, feat_ref, w_ref, out_ref, g_ref):
    i = pl.program_id(0)

    @pl.when(i == 0)
    def _():
        g_ref[...] = lax.dot_general(
            feat_ref[...], w_ref[:, _D:],
            dimension_numbers=(((1,), (1,)), ((), ())),
            preferred_element_type=jnp.float32)

    a = adj_ref[...]
    deg = jnp.sum(a, axis=1, keepdims=True)
    neigh = jnp.dot(a, g_ref[...], preferred_element_type=jnp.float32)
    fb = feat_ref[pl.ds(i * _BM, _BM), :]
    self_term = lax.dot_general(
        fb, w_ref[:, :_D],
        dimension_numbers=(((1,), (1,)), ((), ())),
        preferred_element_type=jnp.float32)
    out_ref[...] = self_term + neigh / (deg + 1.0)


def kernel(adj, features, W):
    return pl.pallas_call(
        _sage_body,
        grid=(_N // _BM,),
        in_specs=[
            pl.BlockSpec((_BM, _N), lambda i: (i, 0)),
            pl.BlockSpec((_N, _D), lambda i: (0, 0)),
            pl.BlockSpec((_D, 2 * _D), lambda i: (0, 0)),
        ],
        out_specs=pl.BlockSpec((_BM, _D), lambda i: (i, 0)),
        out_shape=jax.ShapeDtypeStruct((_N, _D), jnp.float32),
        scratch_shapes=[pltpu.VMEM((_N, _D), jnp.float32)],
    )(adj, features, W)
